# baseline (speedup 1.0000x reference)
import functools

import jax
import jax.numpy as jnp
from jax import lax
from jax.experimental import pallas as pl
from jax.experimental.pallas import tpu as pltpu
from jax.experimental.pallas import tpu_sc as plsc

B = 16384
K = 16
CHUNK = 128


@jax.jit
def _bpr_sc(user, pos_item, neg_item, embedding_user, embedding_item):
    info = plsc.get_sparse_core_info()
    NC, NS = info.num_cores, info.num_subcores
    NW = NC * NS
    b_per_w = B // NW
    n_chunks = b_per_w // CHUNK
    n_user = embedding_user.shape[0]
    n_item = embedding_item.shape[0]

    mesh = plsc.VectorSubcoreMesh(core_axis_name="c", subcore_axis_name="s")

    @functools.partial(
        pl.kernel,
        mesh=mesh,
        compiler_params=pltpu.CompilerParams(
            needs_layout_passes=False, use_tc_tiling_on_sc=False,
            disable_bounds_checks=True),
        out_type=jax.ShapeDtypeStruct((NW, n_chunks, CHUNK), jnp.float32),
        scratch_types=[
            pltpu.VMEM((n_chunks, CHUNK), jnp.int32),
            pltpu.VMEM((n_chunks, CHUNK), jnp.int32),
            pltpu.VMEM((n_chunks, CHUNK), jnp.int32),
            pltpu.VMEM((K, n_chunks, CHUNK), jnp.float32),
            pltpu.VMEM((K, n_chunks, CHUNK), jnp.float32),
            pltpu.VMEM((K, n_chunks, CHUNK), jnp.float32),
            pltpu.VMEM((n_chunks, CHUNK), jnp.float32),
            pltpu.SemaphoreType.DMA,
        ],
    )
    def k(user_hbm, pos_hbm, neg_hbm, eu_hbm, ei_hbm, out_hbm,
          uidx_v, pidx_v, nidx_v, gu_v, gp_v, gn_v, out_v, sem):
        wid = lax.axis_index("s") * NC + lax.axis_index("c")
        pltpu.sync_copy(user_hbm.at[wid], uidx_v)
        pltpu.sync_copy(pos_hbm.at[wid], pidx_v)
        pltpu.sync_copy(neg_hbm.at[wid], nidx_v)

        copies = []
        for comp in range(K):
            src_u = eu_hbm.at[pl.ds(comp * n_user, n_user)]
            src_i = ei_hbm.at[pl.ds(comp * n_item, n_item)]
            for c in range(n_chunks):
                copies.append(pltpu.async_copy(
                    src_u.at[uidx_v.at[c]], gu_v.at[comp, c], sem))
                copies.append(pltpu.async_copy(
                    src_i.at[pidx_v.at[c]], gp_v.at[comp, c], sem))
                copies.append(pltpu.async_copy(
                    src_i.at[nidx_v.at[c]], gn_v.at[comp, c], sem))
        for cp in copies:
            cp.wait()

        def dot_body(c, _):
            for v in range(CHUNK // K):
                sl = pl.ds(v * K, K)
                acc = gu_v[0, c, sl] * (gp_v[0, c, sl] - gn_v[0, c, sl])
                for comp in range(1, K):
                    acc = acc + gu_v[comp, c, sl] * (
                        gp_v[comp, c, sl] - gn_v[comp, c, sl])
                out_v[c, sl] = acc
            return 0

        lax.fori_loop(0, n_chunks, dot_body, 0)
        pltpu.sync_copy(out_v, out_hbm.at[wid])

    out = k(
        user.reshape(NW, n_chunks, CHUNK),
        pos_item.reshape(NW, n_chunks, CHUNK),
        neg_item.reshape(NW, n_chunks, CHUNK),
        embedding_user.T.reshape(K * n_user),
        embedding_item.T.reshape(K * n_item),
    )
    return out.reshape(B)


def kernel(user, pos_item, neg_item, embedding_user, embedding_item):
    return _bpr_sc(user, pos_item, neg_item, embedding_user, embedding_item)
